# baseline (speedup 1.0000x reference)
import functools

import jax
import jax.numpy as jnp
from jax import lax
from jax.experimental import pallas as pl
from jax.experimental.pallas import tpu as pltpu
from jax.experimental.pallas import tpu_sc as plsc

D_MODEL = 64
SCALE = 8.0

NC, NS, LANES = 2, 16, 16
NW = NC * NS
XROWS, XCOLS = 4096, 200
B = XROWS * XCOLS
BPW = B // NW
C = 256
NCHUNK = BPW // C
JB = D_MODEL // LANES
NBUF = 4
LOOKAHEAD = 2

_mesh = plsc.VectorSubcoreMesh(core_axis_name="c", subcore_axis_name="s")


@functools.partial(
    pl.kernel,
    out_type=jax.ShapeDtypeStruct((B, D_MODEL), jnp.float32),
    mesh=_mesh,
    scratch_types=[
        pltpu.VMEM((NCHUNK, C), jnp.int32),
        [pltpu.VMEM((C, D_MODEL), jnp.float32)] * NBUF,
        [pltpu.SemaphoreType.DMA] * NBUF,
        [pltpu.SemaphoreType.DMA] * NBUF,
    ],
    compiler_params=pltpu.CompilerParams(use_tc_tiling_on_sc=False),
)
def _embed_sc(x_hbm, tab_hbm, out_hbm, idx_v, rows, gsems, osems):
    wid = lax.axis_index("s") * NC + lax.axis_index("c")
    base = wid * BPW

    pltpu.sync_copy(x_hbm.at[pl.ds(wid * NCHUNK, NCHUNK)], idx_v)

    def issue_gather(j, b):
        pltpu.async_copy(tab_hbm.at[idx_v.at[j]], rows[b], gsems[b])

    def wait_gather(b):
        pltpu.make_async_copy(tab_hbm.at[idx_v.at[0]], rows[b], gsems[b]).wait()

    def wait_out(b):
        pltpu.make_async_copy(rows[b], out_hbm.at[pl.ds(0, C)], osems[b]).wait()

    def consume(j, b):
        wait_gather(b)

        def scale_row(r, carry):
            for k in range(JB):
                v = rows[b][r, pl.ds(k * LANES, LANES)] * SCALE
                rows[b][r, pl.ds(k * LANES, LANES)] = v
            return carry

        lax.fori_loop(0, C, scale_row, 0)
        pltpu.async_copy(rows[b], out_hbm.at[pl.ds(base + j * C, C)], osems[b])

    def visit(j, b, drain_out):
        bf = (b + LOOKAHEAD) % NBUF
        if drain_out:
            wait_out(bf)
        issue_gather(j + LOOKAHEAD, bf)
        consume(j, b)

    for b in range(LOOKAHEAD):
        issue_gather(b, b)
    for j in range(NBUF):
        visit(j, j, drain_out=(j >= NBUF - LOOKAHEAD))

    def steady(g, carry):
        for b in range(NBUF):
            visit(g * NBUF + b, b, drain_out=True)
        return carry

    lax.fori_loop(1, NCHUNK // NBUF - 1, steady, 0)

    for b in range(NBUF):
        j = NCHUNK - NBUF + b
        if b < NBUF - LOOKAHEAD:
            visit(j, b, drain_out=True)
        else:
            consume(j, b)
    for b in range(NBUF):
        wait_out(b)


def kernel(x, table):
    x_chunks = x.astype(jnp.int32).reshape(NW * NCHUNK, C)
    out_flat = _embed_sc(x_chunks, table)
    return out_flat.reshape(XROWS, XCOLS, D_MODEL)
